# Initial kernel scaffold; baseline (speedup 1.0000x reference)
import jax
import jax.numpy as jnp
from jax.experimental import pallas as pl


def kernel(x, edge_index, batch, W1, b1, W2, b2, Wl, bl):
    raise NotImplementedError("write your pallas kernel here")


# speedup vs baseline: 7.0596x; 7.0596x over previous
import functools

import jax
import jax.numpy as jnp
from jax import lax
from jax.experimental import pallas as pl
from jax.experimental.pallas import tpu as pltpu
from jax.experimental.pallas import tpu_sc as plsc

N = 10000
E = 160000
IN_DIM = 4
HIDDEN = 512
OUT_DIM = 2
G = 64

N_PAD = 10240
E_PAD = 163840
CH = 128
ROWS_T = N_PAD // 16
NB = N_PAD // 512

_MESH = plsc.VectorSubcoreMesh(core_axis_name="c", subcore_axis_name="s")


@functools.partial(
    pl.kernel,
    out_type=jax.ShapeDtypeStruct((2 * N_PAD,), jnp.float32),
    mesh=_MESH,
    scratch_types=[
        pltpu.VMEM((CH,), jnp.int32),
        pltpu.VMEM((CH,), jnp.float32),
        pltpu.VMEM_SHARED((N_PAD,), jnp.float32),
    ],
)
def _sc_deg(dst_hbm, z1_hbm, out_hbm, dst_v, ones_v, acc):
    cid = lax.axis_index("c")
    sid = lax.axis_index("s")

    def fill(i, _):
        ones_v[pl.ds(i * 16, 16)] = jnp.full((16,), 1.0, jnp.float32)
        return 0
    lax.fori_loop(0, CH // 16, fill, 0)

    pltpu.sync_copy(z1_hbm.at[pl.ds(sid * ROWS_T, ROWS_T)],
                    acc.at[pl.ds(sid * ROWS_T, ROWS_T)])
    plsc.subcore_barrier()

    per_tile = E_PAD // 32
    base = (cid * 16 + sid) * per_tile

    def step(c, _):
        pltpu.sync_copy(dst_hbm.at[pl.ds(base + c * CH, CH)], dst_v)
        pltpu.sync_copy(ones_v, acc.at[dst_v], add=True)
        return 0
    lax.fori_loop(0, per_tile // CH, step, 0)
    plsc.subcore_barrier()

    pltpu.sync_copy(acc.at[pl.ds(sid * ROWS_T, ROWS_T)],
                    out_hbm.at[pl.ds(cid * N_PAD + sid * ROWS_T, ROWS_T)])


def _make_sc_agg(num_groups):
    out_rows = (2 if num_groups == 1 else 4) * N_PAD

    @functools.partial(
        pl.kernel,
        out_type=jax.ShapeDtypeStruct((out_rows, 128), jnp.float32),
        mesh=_MESH,
        scratch_types=[
            pltpu.VMEM((CH,), jnp.int32),
            pltpu.VMEM((CH,), jnp.int32),
            pltpu.VMEM((CH, 128), jnp.float32),
            pltpu.VMEM_SHARED((N_PAD, 128), jnp.float32),
        ],
    )
    def agg(src_hbm, dst_hbm, tab_hbm, z128_hbm, out_hbm,
            src_v, dst_v, rows_v, acc):
        cid = lax.axis_index("c")
        sid = lax.axis_index("s")
        if num_groups == 1:
            per_tile = E_PAD // 32
            groups = 1
        else:
            per_tile = E_PAD // 16
            groups = 2

        for j in range(groups):
            pltpu.sync_copy(z128_hbm.at[pl.ds(sid * ROWS_T, ROWS_T)],
                            acc.at[pl.ds(sid * ROWS_T, ROWS_T)])
            plsc.subcore_barrier()

            if num_groups == 1:
                ebase = (cid * 16 + sid) * per_tile
                goff = 0
                orow = cid * N_PAD
            else:
                g = cid * 2 + j
                ebase = sid * per_tile
                goff = g * E_PAD
                orow = g * N_PAD

            def step(c, _):
                eb = ebase + c * CH
                pltpu.sync_copy(src_hbm.at[pl.ds(goff + eb, CH)], src_v)
                pltpu.sync_copy(dst_hbm.at[pl.ds(eb, CH)], dst_v)
                pltpu.sync_copy(tab_hbm.at[src_v], rows_v)
                pltpu.sync_copy(rows_v, acc.at[dst_v], add=True)
                return 0
            lax.fori_loop(0, per_tile // CH, step, 0)
            plsc.subcore_barrier()

            pltpu.sync_copy(acc.at[pl.ds(sid * ROWS_T, ROWS_T)],
                            out_hbm.at[pl.ds(orow + sid * ROWS_T, ROWS_T)])

    return agg


_sc_agg_x = _make_sc_agg(1)
_sc_agg_h = _make_sc_agg(4)


def _tc_prep_body(degt_ref, x_ref, src_ref, dinv_ref, x1_ref, srcg_ref):
    deg = degt_ref[:, 0:1] + degt_ref[:, 1:2] + 1.0
    dinv = lax.rsqrt(deg)
    dinv_ref[...] = dinv
    x1_ref[...] = x_ref[...] * dinv
    s = src_ref[...]
    for g in range(4):
        srcg_ref[g] = s + g * N_PAD


def _tc_prep(degt, x128, src2d):
    return pl.pallas_call(
        _tc_prep_body,
        out_shape=(
            jax.ShapeDtypeStruct((N_PAD, 1), jnp.float32),
            jax.ShapeDtypeStruct((N_PAD, 128), jnp.float32),
            jax.ShapeDtypeStruct((4, E_PAD // 128, 128), jnp.int32),
        ),
    )(degt, x128, src2d)


def _tc_h1_body(aggx_ref, x1_ref, dinv_ref, w1_ref, b1_ref, out_ref):
    dinv = dinv_ref[...]
    agg = aggx_ref[0] + aggx_ref[1] + x1_ref[...]
    t = agg * dinv
    h = jnp.dot(t, w1_ref[...], preferred_element_type=jnp.float32)
    h = jnp.maximum(h + b1_ref[...], 0.0) * dinv
    for g in range(4):
        out_ref[g] = h[:, g * 128:(g + 1) * 128]


def _tc_h1(aggx, x1, dinv, w1p, b1):
    return pl.pallas_call(
        _tc_h1_body,
        grid=(NB,),
        in_specs=[
            pl.BlockSpec((2, 512, 128), lambda i: (0, i, 0)),
            pl.BlockSpec((512, 128), lambda i: (i, 0)),
            pl.BlockSpec((512, 1), lambda i: (i, 0)),
            pl.BlockSpec((128, HIDDEN), lambda i: (0, 0)),
            pl.BlockSpec((1, HIDDEN), lambda i: (0, 0)),
        ],
        out_specs=pl.BlockSpec((4, 512, 128), lambda i: (0, i, 0)),
        out_shape=jax.ShapeDtypeStruct((4, N_PAD, 128), jnp.float32),
        compiler_params=pltpu.CompilerParams(
            dimension_semantics=("arbitrary",)),
    )(aggx, x1, dinv, w1p, b1)


def _tc_out_body(agg_ref, h1g_ref, dinv_ref, batch_ref, w2_ref, b2_ref,
                 wl_ref, bl_ref, out_ref, pool_acc, cnt_acc):
    i = pl.program_id(0)

    @pl.when(i == 0)
    def _():
        pool_acc[...] = jnp.zeros_like(pool_acc)
        cnt_acc[...] = jnp.zeros_like(cnt_acc)

    z = jnp.concatenate(
        [agg_ref[g] + h1g_ref[g] for g in range(4)], axis=1)
    z = z * dinv_ref[...]
    h2 = jnp.dot(z, w2_ref[...], preferred_element_type=jnp.float32)
    h2 = jnp.maximum(h2 + b2_ref[...], 0.0)

    gid = lax.broadcasted_iota(jnp.int32, (512, G), 1)
    p = (batch_ref[...] == gid).astype(jnp.float32)
    pool_acc[...] += lax.dot_general(
        p, h2, (((0,), (0,)), ((), ())),
        preferred_element_type=jnp.float32)
    cnt_acc[...] += lax.dot_general(
        p, jnp.ones((512, 1), jnp.float32), (((0,), (0,)), ((), ())),
        preferred_element_type=jnp.float32)

    @pl.when(i == NB - 1)
    def _():
        pooled = pool_acc[...] / jnp.maximum(cnt_acc[...], 1.0)
        out_ref[...] = (jnp.dot(pooled, wl_ref[...],
                                preferred_element_type=jnp.float32)
                        + bl_ref[...])


def _tc_out(agg, h1g, dinv, batch2d, w2, b2, wl, bl):
    return pl.pallas_call(
        _tc_out_body,
        grid=(NB,),
        in_specs=[
            pl.BlockSpec((4, 512, 128), lambda i: (0, i, 0)),
            pl.BlockSpec((4, 512, 128), lambda i: (0, i, 0)),
            pl.BlockSpec((512, 1), lambda i: (i, 0)),
            pl.BlockSpec((512, 1), lambda i: (i, 0)),
            pl.BlockSpec((HIDDEN, HIDDEN), lambda i: (0, 0)),
            pl.BlockSpec((1, HIDDEN), lambda i: (0, 0)),
            pl.BlockSpec((HIDDEN, OUT_DIM), lambda i: (0, 0)),
            pl.BlockSpec((1, OUT_DIM), lambda i: (0, 0)),
        ],
        out_specs=pl.BlockSpec((G, OUT_DIM), lambda i: (0, 0)),
        out_shape=jax.ShapeDtypeStruct((G, OUT_DIM), jnp.float32),
        scratch_shapes=[
            pltpu.VMEM((G, HIDDEN), jnp.float32),
            pltpu.VMEM((G, 1), jnp.float32),
        ],
        compiler_params=pltpu.CompilerParams(
            dimension_semantics=("arbitrary",)),
    )(agg, h1g, dinv, batch2d, w2, b2, wl, bl)


def kernel(x, edge_index, batch, W1, b1, W2, b2, Wl, bl):
    f32 = jnp.float32
    src = jnp.pad(edge_index[0], (0, E_PAD - E))
    dst = jnp.pad(edge_index[1], (0, E_PAD - E), constant_values=N)
    x128 = jnp.pad(x, ((0, N_PAD - N), (0, 128 - IN_DIM)))
    batch2d = jnp.pad(batch, (0, N_PAD - N), constant_values=G)[:, None]
    w1p = jnp.pad(W1, ((0, 128 - IN_DIM), (0, 0)))
    b1r = b1[None, :]
    b2r = b2[None, :]
    blr = bl[None, :]
    z1 = jnp.zeros((N_PAD,), f32)
    z128 = jnp.zeros((N_PAD, 128), f32)

    degp = _sc_deg(dst, z1)
    degt = jnp.transpose(degp.reshape(2, N_PAD))
    dinv, x1, srcg = _tc_prep(degt, x128, src.reshape(E_PAD // 128, 128))

    aggx = _sc_agg_x(src, dst, x1, z128)
    h1g = _tc_h1(aggx.reshape(2, N_PAD, 128), x1, dinv, w1p, b1r)

    agg1 = _sc_agg_h(srcg.reshape(4 * E_PAD), dst,
                     h1g.reshape(4 * N_PAD, 128), z128)
    return _tc_out(agg1.reshape(4, N_PAD, 128), h1g, dinv, batch2d,
                   W2, b2r, Wl, blr)


# speedup vs baseline: 8.4086x; 1.1911x over previous
import functools

import jax
import jax.numpy as jnp
from jax import lax
from jax.experimental import pallas as pl
from jax.experimental.pallas import tpu as pltpu
from jax.experimental.pallas import tpu_sc as plsc

N = 10000
E = 160000
IN_DIM = 4
HIDDEN = 512
OUT_DIM = 2
G = 64

N_PAD = 10240
E_PAD = 163840
CH = 128
ROWS_T = N_PAD // 16
NB = N_PAD // 512

_MESH = plsc.VectorSubcoreMesh(core_axis_name="c", subcore_axis_name="s")


@functools.partial(
    pl.kernel,
    out_type=jax.ShapeDtypeStruct((2 * N_PAD,), jnp.float32),
    mesh=_MESH,
    scratch_types=[
        pltpu.VMEM((CH,), jnp.int32),
        pltpu.VMEM((CH,), jnp.float32),
        pltpu.VMEM_SHARED((N_PAD,), jnp.float32),
    ],
)
def _sc_deg(dst_hbm, z1_hbm, out_hbm, dst_v, ones_v, acc):
    cid = lax.axis_index("c")
    sid = lax.axis_index("s")

    def fill(i, _):
        ones_v[pl.ds(i * 16, 16)] = jnp.full((16,), 1.0, jnp.float32)
        return 0
    lax.fori_loop(0, CH // 16, fill, 0)

    pltpu.sync_copy(z1_hbm.at[pl.ds(sid * ROWS_T, ROWS_T)],
                    acc.at[pl.ds(sid * ROWS_T, ROWS_T)])
    plsc.subcore_barrier()

    per_tile = E_PAD // 32
    base = (cid * 16 + sid) * per_tile

    def step(c, _):
        pltpu.sync_copy(dst_hbm.at[pl.ds(base + c * CH, CH)], dst_v)
        pltpu.sync_copy(ones_v, acc.at[dst_v], add=True)
        return 0
    lax.fori_loop(0, per_tile // CH, step, 0)
    plsc.subcore_barrier()

    pltpu.sync_copy(acc.at[pl.ds(sid * ROWS_T, ROWS_T)],
                    out_hbm.at[pl.ds(cid * N_PAD + sid * ROWS_T, ROWS_T)])


def _make_sc_agg(num_groups):
    out_rows = (2 if num_groups == 1 else 4) * N_PAD
    per_tile = E_PAD // 32 if num_groups == 1 else E_PAD // 16

    @functools.partial(
        pl.kernel,
        out_type=jax.ShapeDtypeStruct((out_rows, 128), jnp.float32),
        mesh=_MESH,
        scratch_types=[
            pltpu.VMEM((per_tile,), jnp.int32),
            pltpu.VMEM((CH,), jnp.int32),
            pltpu.VMEM((CH,), jnp.int32),
            pltpu.VMEM((CH, 128), jnp.float32),
            pltpu.VMEM((CH, 128), jnp.float32),
            pltpu.SemaphoreType.DMA,
            pltpu.SemaphoreType.DMA,
            pltpu.SemaphoreType.DMA,
            pltpu.SemaphoreType.DMA,
            pltpu.VMEM_SHARED((N_PAD, 128), jnp.float32),
        ],
    )
    def agg(src_hbm, dst_hbm, tab_hbm, z128_hbm, out_hbm,
            src_all, dstb0, dstb1, rows0, rows1,
            gsem0, gsem1, dsem0, dsem1, acc):
        cid = lax.axis_index("c")
        sid = lax.axis_index("s")
        groups = 1 if num_groups == 1 else 2

        for j in range(groups):
            pltpu.sync_copy(z128_hbm.at[pl.ds(sid * ROWS_T, ROWS_T)],
                            acc.at[pl.ds(sid * ROWS_T, ROWS_T)])

            if num_groups == 1:
                ebase = (cid * 16 + sid) * per_tile
                goff = 0
                orow = cid * N_PAD
            else:
                g = cid * 2 + j
                ebase = sid * per_tile
                goff = g * E_PAD
                orow = g * N_PAD

            pltpu.sync_copy(src_hbm.at[pl.ds(goff + ebase, per_tile)],
                            src_all)
            plsc.subcore_barrier()

            def step2(c2, _):
                c0 = 2 * c2
                e0 = ebase + c0 * CH
                d0 = pltpu.async_copy(dst_hbm.at[pl.ds(e0, CH)], dstb0,
                                      dsem0)
                d1 = pltpu.async_copy(dst_hbm.at[pl.ds(e0 + CH, CH)], dstb1,
                                      dsem1)
                g0 = pltpu.async_copy(
                    tab_hbm.at[src_all.at[pl.ds(c0 * CH, CH)]], rows0, gsem0)
                g1 = pltpu.async_copy(
                    tab_hbm.at[src_all.at[pl.ds(c0 * CH + CH, CH)]], rows1,
                    gsem1)
                g0.wait()
                d0.wait()
                pltpu.sync_copy(rows0, acc.at[dstb0], add=True)
                g1.wait()
                d1.wait()
                pltpu.sync_copy(rows1, acc.at[dstb1], add=True)
                return 0
            lax.fori_loop(0, per_tile // (2 * CH), step2, 0)
            plsc.subcore_barrier()

            pltpu.sync_copy(acc.at[pl.ds(sid * ROWS_T, ROWS_T)],
                            out_hbm.at[pl.ds(orow + sid * ROWS_T, ROWS_T)])

    return agg


_sc_agg_x = _make_sc_agg(1)
_sc_agg_h = _make_sc_agg(4)


def _tc_prep_body(degt_ref, x_ref, src_ref, dinv_ref, x1_ref, srcg_ref):
    deg = degt_ref[:, 0:1] + degt_ref[:, 1:2] + 1.0
    dinv = lax.rsqrt(deg)
    dinv_ref[...] = dinv
    x1_ref[...] = x_ref[...] * dinv
    s = src_ref[...]
    for g in range(4):
        srcg_ref[g] = s + g * N_PAD


def _tc_prep(degt, x128, src2d):
    return pl.pallas_call(
        _tc_prep_body,
        out_shape=(
            jax.ShapeDtypeStruct((N_PAD, 1), jnp.float32),
            jax.ShapeDtypeStruct((N_PAD, 128), jnp.float32),
            jax.ShapeDtypeStruct((4, E_PAD // 128, 128), jnp.int32),
        ),
    )(degt, x128, src2d)


def _tc_h1_body(aggx_ref, x1_ref, dinv_ref, w1_ref, b1_ref, out_ref):
    dinv = dinv_ref[...]
    agg = aggx_ref[0] + aggx_ref[1] + x1_ref[...]
    t = agg * dinv
    h = jnp.dot(t, w1_ref[...], preferred_element_type=jnp.float32)
    h = jnp.maximum(h + b1_ref[...], 0.0) * dinv
    for g in range(4):
        out_ref[g] = h[:, g * 128:(g + 1) * 128]


def _tc_h1(aggx, x1, dinv, w1p, b1):
    return pl.pallas_call(
        _tc_h1_body,
        grid=(NB,),
        in_specs=[
            pl.BlockSpec((2, 512, 128), lambda i: (0, i, 0)),
            pl.BlockSpec((512, 128), lambda i: (i, 0)),
            pl.BlockSpec((512, 1), lambda i: (i, 0)),
            pl.BlockSpec((128, HIDDEN), lambda i: (0, 0)),
            pl.BlockSpec((1, HIDDEN), lambda i: (0, 0)),
        ],
        out_specs=pl.BlockSpec((4, 512, 128), lambda i: (0, i, 0)),
        out_shape=jax.ShapeDtypeStruct((4, N_PAD, 128), jnp.float32),
        compiler_params=pltpu.CompilerParams(
            dimension_semantics=("arbitrary",)),
    )(aggx, x1, dinv, w1p, b1)


def _tc_out_body(agg_ref, h1g_ref, dinv_ref, batch_ref, w2_ref, b2_ref,
                 wl_ref, bl_ref, out_ref, pool_acc, cnt_acc):
    i = pl.program_id(0)

    @pl.when(i == 0)
    def _():
        pool_acc[...] = jnp.zeros_like(pool_acc)
        cnt_acc[...] = jnp.zeros_like(cnt_acc)

    z = jnp.concatenate(
        [agg_ref[g] + h1g_ref[g] for g in range(4)], axis=1)
    z = z * dinv_ref[...]
    h2 = jnp.dot(z, w2_ref[...], preferred_element_type=jnp.float32)
    h2 = jnp.maximum(h2 + b2_ref[...], 0.0)

    gid = lax.broadcasted_iota(jnp.int32, (512, G), 1)
    p = (batch_ref[...] == gid).astype(jnp.float32)
    pool_acc[...] += lax.dot_general(
        p, h2, (((0,), (0,)), ((), ())),
        preferred_element_type=jnp.float32)
    cnt_acc[...] += lax.dot_general(
        p, jnp.ones((512, 1), jnp.float32), (((0,), (0,)), ((), ())),
        preferred_element_type=jnp.float32)

    @pl.when(i == NB - 1)
    def _():
        pooled = pool_acc[...] / jnp.maximum(cnt_acc[...], 1.0)
        out_ref[...] = (jnp.dot(pooled, wl_ref[...],
                                preferred_element_type=jnp.float32)
                        + bl_ref[...])


def _tc_out(agg, h1g, dinv, batch2d, w2, b2, wl, bl):
    return pl.pallas_call(
        _tc_out_body,
        grid=(NB,),
        in_specs=[
            pl.BlockSpec((4, 512, 128), lambda i: (0, i, 0)),
            pl.BlockSpec((4, 512, 128), lambda i: (0, i, 0)),
            pl.BlockSpec((512, 1), lambda i: (i, 0)),
            pl.BlockSpec((512, 1), lambda i: (i, 0)),
            pl.BlockSpec((HIDDEN, HIDDEN), lambda i: (0, 0)),
            pl.BlockSpec((1, HIDDEN), lambda i: (0, 0)),
            pl.BlockSpec((HIDDEN, OUT_DIM), lambda i: (0, 0)),
            pl.BlockSpec((1, OUT_DIM), lambda i: (0, 0)),
        ],
        out_specs=pl.BlockSpec((G, OUT_DIM), lambda i: (0, 0)),
        out_shape=jax.ShapeDtypeStruct((G, OUT_DIM), jnp.float32),
        scratch_shapes=[
            pltpu.VMEM((G, HIDDEN), jnp.float32),
            pltpu.VMEM((G, 1), jnp.float32),
        ],
        compiler_params=pltpu.CompilerParams(
            dimension_semantics=("arbitrary",)),
    )(agg, h1g, dinv, batch2d, w2, b2, wl, bl)


def kernel(x, edge_index, batch, W1, b1, W2, b2, Wl, bl):
    f32 = jnp.float32
    src = jnp.pad(edge_index[0], (0, E_PAD - E))
    dst = jnp.pad(edge_index[1], (0, E_PAD - E), constant_values=N)
    x128 = jnp.pad(x, ((0, N_PAD - N), (0, 128 - IN_DIM)))
    batch2d = jnp.pad(batch, (0, N_PAD - N), constant_values=G)[:, None]
    w1p = jnp.pad(W1, ((0, 128 - IN_DIM), (0, 0)))
    b1r = b1[None, :]
    b2r = b2[None, :]
    blr = bl[None, :]
    z1 = jnp.zeros((N_PAD,), f32)
    z128 = jnp.zeros((N_PAD, 128), f32)

    degp = _sc_deg(dst, z1)
    degt = jnp.transpose(degp.reshape(2, N_PAD))
    dinv, x1, srcg = _tc_prep(degt, x128, src.reshape(E_PAD // 128, 128))

    aggx = _sc_agg_x(src, dst, x1, z128)
    h1g = _tc_h1(aggx.reshape(2, N_PAD, 128), x1, dinv, w1p, b1r)

    agg1 = _sc_agg_h(srcg.reshape(4 * E_PAD), dst,
                     h1g.reshape(4 * N_PAD, 128), z128)
    return _tc_out(agg1.reshape(4, N_PAD, 128), h1g, dinv, batch2d,
                   W2, b2r, Wl, blr)


# speedup vs baseline: 8.8772x; 1.0557x over previous
import functools

import jax
import jax.numpy as jnp
from jax import lax
from jax.experimental import pallas as pl
from jax.experimental.pallas import tpu as pltpu
from jax.experimental.pallas import tpu_sc as plsc

N = 10000
E = 160000
IN_DIM = 4
HIDDEN = 512
OUT_DIM = 2
G = 64

N_PAD = 10240
E_PAD = 163840
CH = 128
ROWS_T = N_PAD // 16
NB = N_PAD // 512

_MESH = plsc.VectorSubcoreMesh(core_axis_name="c", subcore_axis_name="s")


@functools.partial(
    pl.kernel,
    out_type=jax.ShapeDtypeStruct((2 * N_PAD,), jnp.float32),
    mesh=_MESH,
    scratch_types=[
        pltpu.VMEM((CH,), jnp.int32),
        pltpu.VMEM((CH,), jnp.float32),
        pltpu.VMEM_SHARED((N_PAD,), jnp.float32),
    ],
)
def _sc_deg(dst_hbm, z1_hbm, out_hbm, dst_v, ones_v, acc):
    cid = lax.axis_index("c")
    sid = lax.axis_index("s")

    def fill(i, _):
        ones_v[pl.ds(i * 16, 16)] = jnp.full((16,), 1.0, jnp.float32)
        return 0
    lax.fori_loop(0, CH // 16, fill, 0)

    pltpu.sync_copy(z1_hbm.at[pl.ds(sid * ROWS_T, ROWS_T)],
                    acc.at[pl.ds(sid * ROWS_T, ROWS_T)])
    plsc.subcore_barrier()

    per_tile = E_PAD // 32
    base = (cid * 16 + sid) * per_tile

    def step(c, _):
        pltpu.sync_copy(dst_hbm.at[pl.ds(base + c * CH, CH)], dst_v)
        pltpu.sync_copy(ones_v, acc.at[dst_v], add=True)
        return 0
    lax.fori_loop(0, per_tile // CH, step, 0)
    plsc.subcore_barrier()

    pltpu.sync_copy(acc.at[pl.ds(sid * ROWS_T, ROWS_T)],
                    out_hbm.at[pl.ds(cid * N_PAD + sid * ROWS_T, ROWS_T)])


def _make_sc_agg(num_groups):
    out_rows = (2 if num_groups == 1 else 4) * N_PAD
    per_tile = E_PAD // 32 if num_groups == 1 else E_PAD // 16

    NBUF = 2
    steps = per_tile // CH
    iters = steps // NBUF

    @functools.partial(
        pl.kernel,
        out_type=jax.ShapeDtypeStruct((out_rows, 128), jnp.float32),
        mesh=_MESH,
        scratch_types=(
            [pltpu.VMEM((per_tile,), jnp.int32)]
            + [pltpu.VMEM((CH,), jnp.int32) for _ in range(NBUF)]
            + [pltpu.VMEM((CH, 128), jnp.float32) for _ in range(NBUF)]
            + [pltpu.SemaphoreType.DMA for _ in range(3 * NBUF)]
            + [pltpu.VMEM_SHARED((N_PAD, 128), jnp.float32)]
        ),
    )
    def agg(src_hbm, dst_hbm, tab_hbm, z128_hbm, out_hbm, *refs):
        src_all = refs[0]
        dstb = refs[1:1 + NBUF]
        rows = refs[1 + NBUF:1 + 2 * NBUF]
        dsem = refs[1 + 2 * NBUF:1 + 3 * NBUF]
        gsem = refs[1 + 3 * NBUF:1 + 4 * NBUF]
        ssem = refs[1 + 4 * NBUF:1 + 5 * NBUF]
        acc = refs[1 + 5 * NBUF]
        cid = lax.axis_index("c")
        sid = lax.axis_index("s")
        groups = 1 if num_groups == 1 else 2

        for j in range(groups):
            pltpu.sync_copy(z128_hbm.at[pl.ds(sid * ROWS_T, ROWS_T)],
                            acc.at[pl.ds(sid * ROWS_T, ROWS_T)])

            if num_groups == 1:
                ebase = (cid * 16 + sid) * per_tile
                goff = 0
                orow = cid * N_PAD
            else:
                g = cid * 2 + j
                ebase = sid * per_tile
                goff = g * E_PAD
                orow = g * N_PAD

            pltpu.sync_copy(src_hbm.at[pl.ds(goff + ebase, per_tile)],
                            src_all)
            plsc.subcore_barrier()

            def fire(c, k):
                pltpu.async_copy(dst_hbm.at[pl.ds(ebase + c * CH, CH)],
                                 dstb[k], dsem[k])
                pltpu.async_copy(tab_hbm.at[src_all.at[pl.ds(c * CH, CH)]],
                                 rows[k], gsem[k])

            def wait_fire(c, k):
                pltpu.make_async_copy(dst_hbm.at[pl.ds(ebase + c * CH, CH)],
                                      dstb[k], dsem[k]).wait()
                pltpu.make_async_copy(
                    tab_hbm.at[src_all.at[pl.ds(c * CH, CH)]],
                    rows[k], gsem[k]).wait()

            def scatter(k):
                pltpu.async_copy(rows[k], acc.at[dstb[k]], ssem[k],
                                 add=True)

            def wait_scatter(k):
                pltpu.make_async_copy(rows[k], acc.at[dstb[k]],
                                      ssem[k]).wait()

            for k in range(NBUF):
                fire(jnp.int32(k), k)
            for k in range(NBUF):
                wait_fire(jnp.int32(k), k)
                scatter(k)

            def body(c2, _):
                cb = c2 * NBUF
                for k in range(NBUF):
                    wait_scatter(k)
                    fire(cb + k, k)
                for k in range(NBUF):
                    wait_fire(cb + k, k)
                    scatter(k)
                return 0
            lax.fori_loop(1, iters, body, 0)
            for k in range(NBUF):
                wait_scatter(k)
            plsc.subcore_barrier()

            pltpu.sync_copy(acc.at[pl.ds(sid * ROWS_T, ROWS_T)],
                            out_hbm.at[pl.ds(orow + sid * ROWS_T, ROWS_T)])

    return agg


_sc_agg_x = _make_sc_agg(1)
_sc_agg_h = _make_sc_agg(4)


def _tc_prep_body(degt_ref, x_ref, src_ref, dinv_ref, x1_ref, srcg_ref):
    deg = degt_ref[:, 0:1] + degt_ref[:, 1:2] + 1.0
    dinv = lax.rsqrt(deg)
    dinv_ref[...] = dinv
    x1_ref[...] = x_ref[...] * dinv
    s = src_ref[...]
    for g in range(4):
        srcg_ref[g] = s + g * N_PAD


def _tc_prep(degt, x128, src2d):
    return pl.pallas_call(
        _tc_prep_body,
        out_shape=(
            jax.ShapeDtypeStruct((N_PAD, 1), jnp.float32),
            jax.ShapeDtypeStruct((N_PAD, 128), jnp.float32),
            jax.ShapeDtypeStruct((4, E_PAD // 128, 128), jnp.int32),
        ),
    )(degt, x128, src2d)


def _tc_h1_body(aggx_ref, x1_ref, dinv_ref, w1_ref, b1_ref, out_ref):
    dinv = dinv_ref[...]
    agg = aggx_ref[0] + aggx_ref[1] + x1_ref[...]
    t = agg * dinv
    h = jnp.dot(t, w1_ref[...], preferred_element_type=jnp.float32)
    h = jnp.maximum(h + b1_ref[...], 0.0) * dinv
    for g in range(4):
        out_ref[g] = h[:, g * 128:(g + 1) * 128]


def _tc_h1(aggx, x1, dinv, w1p, b1):
    return pl.pallas_call(
        _tc_h1_body,
        grid=(NB,),
        in_specs=[
            pl.BlockSpec((2, 512, 128), lambda i: (0, i, 0)),
            pl.BlockSpec((512, 128), lambda i: (i, 0)),
            pl.BlockSpec((512, 1), lambda i: (i, 0)),
            pl.BlockSpec((128, HIDDEN), lambda i: (0, 0)),
            pl.BlockSpec((1, HIDDEN), lambda i: (0, 0)),
        ],
        out_specs=pl.BlockSpec((4, 512, 128), lambda i: (0, i, 0)),
        out_shape=jax.ShapeDtypeStruct((4, N_PAD, 128), jnp.float32),
        compiler_params=pltpu.CompilerParams(
            dimension_semantics=("arbitrary",)),
    )(aggx, x1, dinv, w1p, b1)


def _tc_out_body(agg_ref, h1g_ref, dinv_ref, batch_ref, w2_ref, b2_ref,
                 wl_ref, bl_ref, out_ref, pool_acc, cnt_acc):
    i = pl.program_id(0)

    @pl.when(i == 0)
    def _():
        pool_acc[...] = jnp.zeros_like(pool_acc)
        cnt_acc[...] = jnp.zeros_like(cnt_acc)

    z = jnp.concatenate(
        [agg_ref[g] + h1g_ref[g] for g in range(4)], axis=1)
    z = z * dinv_ref[...]
    h2 = jnp.dot(z, w2_ref[...], preferred_element_type=jnp.float32)
    h2 = jnp.maximum(h2 + b2_ref[...], 0.0)

    gid = lax.broadcasted_iota(jnp.int32, (512, G), 1)
    p = (batch_ref[...] == gid).astype(jnp.float32)
    pool_acc[...] += lax.dot_general(
        p, h2, (((0,), (0,)), ((), ())),
        preferred_element_type=jnp.float32)
    cnt_acc[...] += lax.dot_general(
        p, jnp.ones((512, 1), jnp.float32), (((0,), (0,)), ((), ())),
        preferred_element_type=jnp.float32)

    @pl.when(i == NB - 1)
    def _():
        pooled = pool_acc[...] / jnp.maximum(cnt_acc[...], 1.0)
        out_ref[...] = (jnp.dot(pooled, wl_ref[...],
                                preferred_element_type=jnp.float32)
                        + bl_ref[...])


def _tc_out(agg, h1g, dinv, batch2d, w2, b2, wl, bl):
    return pl.pallas_call(
        _tc_out_body,
        grid=(NB,),
        in_specs=[
            pl.BlockSpec((4, 512, 128), lambda i: (0, i, 0)),
            pl.BlockSpec((4, 512, 128), lambda i: (0, i, 0)),
            pl.BlockSpec((512, 1), lambda i: (i, 0)),
            pl.BlockSpec((512, 1), lambda i: (i, 0)),
            pl.BlockSpec((HIDDEN, HIDDEN), lambda i: (0, 0)),
            pl.BlockSpec((1, HIDDEN), lambda i: (0, 0)),
            pl.BlockSpec((HIDDEN, OUT_DIM), lambda i: (0, 0)),
            pl.BlockSpec((1, OUT_DIM), lambda i: (0, 0)),
        ],
        out_specs=pl.BlockSpec((G, OUT_DIM), lambda i: (0, 0)),
        out_shape=jax.ShapeDtypeStruct((G, OUT_DIM), jnp.float32),
        scratch_shapes=[
            pltpu.VMEM((G, HIDDEN), jnp.float32),
            pltpu.VMEM((G, 1), jnp.float32),
        ],
        compiler_params=pltpu.CompilerParams(
            dimension_semantics=("arbitrary",)),
    )(agg, h1g, dinv, batch2d, w2, b2, wl, bl)


def kernel(x, edge_index, batch, W1, b1, W2, b2, Wl, bl):
    f32 = jnp.float32
    src = jnp.pad(edge_index[0], (0, E_PAD - E))
    dst = jnp.pad(edge_index[1], (0, E_PAD - E), constant_values=N)
    x128 = jnp.pad(x, ((0, N_PAD - N), (0, 128 - IN_DIM)))
    batch2d = jnp.pad(batch, (0, N_PAD - N), constant_values=G)[:, None]
    w1p = jnp.pad(W1, ((0, 128 - IN_DIM), (0, 0)))
    b1r = b1[None, :]
    b2r = b2[None, :]
    blr = bl[None, :]
    z1 = jnp.zeros((N_PAD,), f32)
    z128 = jnp.zeros((N_PAD, 128), f32)

    degp = _sc_deg(dst, z1)
    degt = jnp.transpose(degp.reshape(2, N_PAD))
    dinv, x1, srcg = _tc_prep(degt, x128, src.reshape(E_PAD // 128, 128))

    aggx = _sc_agg_x(src, dst, x1, z128)
    h1g = _tc_h1(aggx.reshape(2, N_PAD, 128), x1, dinv, w1p, b1r)

    agg1 = _sc_agg_h(srcg.reshape(4 * E_PAD), dst,
                     h1g.reshape(4 * N_PAD, 128), z128)
    return _tc_out(agg1.reshape(4, N_PAD, 128), h1g, dinv, batch2d,
                   W2, b2r, Wl, blr)


# speedup vs baseline: 9.2019x; 1.0366x over previous
import functools

import jax
import jax.numpy as jnp
from jax import lax
from jax.experimental import pallas as pl
from jax.experimental.pallas import tpu as pltpu
from jax.experimental.pallas import tpu_sc as plsc

N = 10000
E = 160000
IN_DIM = 4
HIDDEN = 512
OUT_DIM = 2
G = 64

N_PAD = 10240
E_PAD = 163840
CH = 128
ROWS_T = N_PAD // 16
NB = N_PAD // 512

_MESH = plsc.VectorSubcoreMesh(core_axis_name="c", subcore_axis_name="s")


@functools.partial(
    pl.kernel,
    out_type=jax.ShapeDtypeStruct((2 * N_PAD,), jnp.float32),
    mesh=_MESH,
    scratch_types=[
        pltpu.VMEM((CH,), jnp.int32),
        pltpu.VMEM((CH,), jnp.float32),
        pltpu.VMEM_SHARED((N_PAD,), jnp.float32),
    ],
)
def _sc_deg(dst_hbm, z1_hbm, out_hbm, dst_v, ones_v, acc):
    cid = lax.axis_index("c")
    sid = lax.axis_index("s")

    def fill(i, _):
        ones_v[pl.ds(i * 16, 16)] = jnp.full((16,), 1.0, jnp.float32)
        return 0
    lax.fori_loop(0, CH // 16, fill, 0)

    pltpu.sync_copy(z1_hbm.at[pl.ds(sid * ROWS_T, ROWS_T)],
                    acc.at[pl.ds(sid * ROWS_T, ROWS_T)])
    plsc.subcore_barrier()

    per_tile = E_PAD // 32
    base = (cid * 16 + sid) * per_tile

    def step(c, _):
        pltpu.sync_copy(dst_hbm.at[pl.ds(base + c * CH, CH)], dst_v)
        pltpu.sync_copy(ones_v, acc.at[dst_v], add=True)
        return 0
    lax.fori_loop(0, per_tile // CH, step, 0)
    plsc.subcore_barrier()

    pltpu.sync_copy(acc.at[pl.ds(sid * ROWS_T, ROWS_T)],
                    out_hbm.at[pl.ds(cid * N_PAD + sid * ROWS_T, ROWS_T)])


def _make_sc_agg(num_groups):
    out_rows = (2 if num_groups == 1 else 4) * N_PAD
    per_tile = E_PAD // 32 if num_groups == 1 else E_PAD // 16

    NBUF = 4
    CHA = 64
    steps = per_tile // CHA
    iters = steps // NBUF

    @functools.partial(
        pl.kernel,
        out_type=jax.ShapeDtypeStruct((out_rows, 128), jnp.float32),
        mesh=_MESH,
        scratch_types=(
            [pltpu.VMEM((per_tile,), jnp.int32)]
            + [pltpu.VMEM((CHA,), jnp.int32) for _ in range(NBUF)]
            + [pltpu.VMEM((CHA, 128), jnp.float32) for _ in range(NBUF)]
            + [pltpu.SemaphoreType.DMA for _ in range(3 * NBUF)]
            + [pltpu.VMEM_SHARED((N_PAD, 128), jnp.float32)]
        ),
    )
    def agg(src_hbm, dst_hbm, tab_hbm, z128_hbm, out_hbm, *refs):
        src_all = refs[0]
        dstb = refs[1:1 + NBUF]
        rows = refs[1 + NBUF:1 + 2 * NBUF]
        dsem = refs[1 + 2 * NBUF:1 + 3 * NBUF]
        gsem = refs[1 + 3 * NBUF:1 + 4 * NBUF]
        ssem = refs[1 + 4 * NBUF:1 + 5 * NBUF]
        acc = refs[1 + 5 * NBUF]
        cid = lax.axis_index("c")
        sid = lax.axis_index("s")
        groups = 1 if num_groups == 1 else 2

        for j in range(groups):
            pltpu.sync_copy(z128_hbm.at[pl.ds(sid * ROWS_T, ROWS_T)],
                            acc.at[pl.ds(sid * ROWS_T, ROWS_T)])

            if num_groups == 1:
                ebase = (cid * 16 + sid) * per_tile
                goff = 0
                orow = cid * N_PAD
            else:
                g = cid * 2 + j
                ebase = sid * per_tile
                goff = g * E_PAD
                orow = g * N_PAD

            pltpu.sync_copy(src_hbm.at[pl.ds(goff + ebase, per_tile)],
                            src_all)
            plsc.subcore_barrier()

            def fire(c, k):
                pltpu.async_copy(dst_hbm.at[pl.ds(ebase + c * CHA, CHA)],
                                 dstb[k], dsem[k])
                pltpu.async_copy(tab_hbm.at[src_all.at[pl.ds(c * CHA, CHA)]],
                                 rows[k], gsem[k])

            def wait_fire(c, k):
                pltpu.make_async_copy(dst_hbm.at[pl.ds(ebase + c * CHA, CHA)],
                                      dstb[k], dsem[k]).wait()
                pltpu.make_async_copy(
                    tab_hbm.at[src_all.at[pl.ds(c * CHA, CHA)]],
                    rows[k], gsem[k]).wait()

            def scatter(k):
                pltpu.async_copy(rows[k], acc.at[dstb[k]], ssem[k],
                                 add=True)

            def wait_scatter(k):
                pltpu.make_async_copy(rows[k], acc.at[dstb[k]],
                                      ssem[k]).wait()

            for k in range(NBUF):
                fire(jnp.int32(k), k)
            for k in range(NBUF):
                wait_fire(jnp.int32(k), k)
                scatter(k)

            def body(c2, _):
                cb = c2 * NBUF
                for k in range(NBUF):
                    wait_scatter(k)
                    fire(cb + k, k)
                for k in range(NBUF):
                    wait_fire(cb + k, k)
                    scatter(k)
                return 0
            lax.fori_loop(1, iters, body, 0)
            for k in range(NBUF):
                wait_scatter(k)
            plsc.subcore_barrier()

            pltpu.sync_copy(acc.at[pl.ds(sid * ROWS_T, ROWS_T)],
                            out_hbm.at[pl.ds(orow + sid * ROWS_T, ROWS_T)])

    return agg


_sc_agg_x = _make_sc_agg(1)
_sc_agg_h = _make_sc_agg(4)


def _tc_prep_body(degt_ref, x_ref, src_ref, dinv_ref, x1_ref, srcg_ref):
    deg = degt_ref[:, 0:1] + degt_ref[:, 1:2] + 1.0
    dinv = lax.rsqrt(deg)
    dinv_ref[...] = dinv
    x1_ref[...] = x_ref[...] * dinv
    s = src_ref[...]
    for g in range(4):
        srcg_ref[g] = s + g * N_PAD


def _tc_prep(degt, x128, src2d):
    return pl.pallas_call(
        _tc_prep_body,
        out_shape=(
            jax.ShapeDtypeStruct((N_PAD, 1), jnp.float32),
            jax.ShapeDtypeStruct((N_PAD, 128), jnp.float32),
            jax.ShapeDtypeStruct((4, E_PAD // 128, 128), jnp.int32),
        ),
    )(degt, x128, src2d)


def _tc_h1_body(aggx_ref, x1_ref, dinv_ref, w1_ref, b1_ref, out_ref):
    dinv = dinv_ref[...]
    agg = aggx_ref[0] + aggx_ref[1] + x1_ref[...]
    t = agg * dinv
    h = jnp.dot(t, w1_ref[...], preferred_element_type=jnp.float32)
    h = jnp.maximum(h + b1_ref[...], 0.0) * dinv
    for g in range(4):
        out_ref[g] = h[:, g * 128:(g + 1) * 128]


def _tc_h1(aggx, x1, dinv, w1p, b1):
    return pl.pallas_call(
        _tc_h1_body,
        grid=(NB,),
        in_specs=[
            pl.BlockSpec((2, 512, 128), lambda i: (0, i, 0)),
            pl.BlockSpec((512, 128), lambda i: (i, 0)),
            pl.BlockSpec((512, 1), lambda i: (i, 0)),
            pl.BlockSpec((128, HIDDEN), lambda i: (0, 0)),
            pl.BlockSpec((1, HIDDEN), lambda i: (0, 0)),
        ],
        out_specs=pl.BlockSpec((4, 512, 128), lambda i: (0, i, 0)),
        out_shape=jax.ShapeDtypeStruct((4, N_PAD, 128), jnp.float32),
        compiler_params=pltpu.CompilerParams(
            dimension_semantics=("arbitrary",)),
    )(aggx, x1, dinv, w1p, b1)


def _tc_out_body(agg_ref, h1g_ref, dinv_ref, batch_ref, w2_ref, b2_ref,
                 wl_ref, bl_ref, out_ref, pool_acc, cnt_acc):
    i = pl.program_id(0)

    @pl.when(i == 0)
    def _():
        pool_acc[...] = jnp.zeros_like(pool_acc)
        cnt_acc[...] = jnp.zeros_like(cnt_acc)

    z = jnp.concatenate(
        [agg_ref[g] + h1g_ref[g] for g in range(4)], axis=1)
    z = z * dinv_ref[...]
    h2 = jnp.dot(z, w2_ref[...], preferred_element_type=jnp.float32)
    h2 = jnp.maximum(h2 + b2_ref[...], 0.0)

    gid = lax.broadcasted_iota(jnp.int32, (512, G), 1)
    p = (batch_ref[...] == gid).astype(jnp.float32)
    pool_acc[...] += lax.dot_general(
        p, h2, (((0,), (0,)), ((), ())),
        preferred_element_type=jnp.float32)
    cnt_acc[...] += lax.dot_general(
        p, jnp.ones((512, 1), jnp.float32), (((0,), (0,)), ((), ())),
        preferred_element_type=jnp.float32)

    @pl.when(i == NB - 1)
    def _():
        pooled = pool_acc[...] / jnp.maximum(cnt_acc[...], 1.0)
        out_ref[...] = (jnp.dot(pooled, wl_ref[...],
                                preferred_element_type=jnp.float32)
                        + bl_ref[...])


def _tc_out(agg, h1g, dinv, batch2d, w2, b2, wl, bl):
    return pl.pallas_call(
        _tc_out_body,
        grid=(NB,),
        in_specs=[
            pl.BlockSpec((4, 512, 128), lambda i: (0, i, 0)),
            pl.BlockSpec((4, 512, 128), lambda i: (0, i, 0)),
            pl.BlockSpec((512, 1), lambda i: (i, 0)),
            pl.BlockSpec((512, 1), lambda i: (i, 0)),
            pl.BlockSpec((HIDDEN, HIDDEN), lambda i: (0, 0)),
            pl.BlockSpec((1, HIDDEN), lambda i: (0, 0)),
            pl.BlockSpec((HIDDEN, OUT_DIM), lambda i: (0, 0)),
            pl.BlockSpec((1, OUT_DIM), lambda i: (0, 0)),
        ],
        out_specs=pl.BlockSpec((G, OUT_DIM), lambda i: (0, 0)),
        out_shape=jax.ShapeDtypeStruct((G, OUT_DIM), jnp.float32),
        scratch_shapes=[
            pltpu.VMEM((G, HIDDEN), jnp.float32),
            pltpu.VMEM((G, 1), jnp.float32),
        ],
        compiler_params=pltpu.CompilerParams(
            dimension_semantics=("arbitrary",)),
    )(agg, h1g, dinv, batch2d, w2, b2, wl, bl)


def kernel(x, edge_index, batch, W1, b1, W2, b2, Wl, bl):
    f32 = jnp.float32
    src = jnp.pad(edge_index[0], (0, E_PAD - E))
    dst = jnp.pad(edge_index[1], (0, E_PAD - E), constant_values=N)
    x128 = jnp.pad(x, ((0, N_PAD - N), (0, 128 - IN_DIM)))
    batch2d = jnp.pad(batch, (0, N_PAD - N), constant_values=G)[:, None]
    w1p = jnp.pad(W1, ((0, 128 - IN_DIM), (0, 0)))
    b1r = b1[None, :]
    b2r = b2[None, :]
    blr = bl[None, :]
    z1 = jnp.zeros((N_PAD,), f32)
    z128 = jnp.zeros((N_PAD, 128), f32)

    degp = _sc_deg(dst, z1)
    degt = jnp.transpose(degp.reshape(2, N_PAD))
    dinv, x1, srcg = _tc_prep(degt, x128, src.reshape(E_PAD // 128, 128))

    aggx = _sc_agg_x(src, dst, x1, z128)
    h1g = _tc_h1(aggx.reshape(2, N_PAD, 128), x1, dinv, w1p, b1r)

    agg1 = _sc_agg_h(srcg.reshape(4 * E_PAD), dst,
                     h1g.reshape(4 * N_PAD, 128), z128)
    return _tc_out(agg1.reshape(4, N_PAD, 128), h1g, dinv, batch2d,
                   W2, b2r, Wl, blr)


# speedup vs baseline: 9.2128x; 1.0012x over previous
import functools

import jax
import jax.numpy as jnp
from jax import lax
from jax.experimental import pallas as pl
from jax.experimental.pallas import tpu as pltpu
from jax.experimental.pallas import tpu_sc as plsc

N = 10000
E = 160000
IN_DIM = 4
HIDDEN = 512
OUT_DIM = 2
G = 64

N_PAD = 10240
E_PAD = 163840
CH = 128
ROWS_T = N_PAD // 16
NB = N_PAD // 512

_MESH = plsc.VectorSubcoreMesh(core_axis_name="c", subcore_axis_name="s")


@functools.partial(
    pl.kernel,
    out_type=jax.ShapeDtypeStruct((2 * N_PAD,), jnp.float32),
    mesh=_MESH,
    scratch_types=[
        pltpu.VMEM((CH,), jnp.int32),
        pltpu.VMEM((CH,), jnp.float32),
        pltpu.VMEM_SHARED((N_PAD,), jnp.float32),
    ],
)
def _sc_deg(dst_hbm, z1_hbm, out_hbm, dst_v, ones_v, acc):
    cid = lax.axis_index("c")
    sid = lax.axis_index("s")

    def fill(i, _):
        ones_v[pl.ds(i * 16, 16)] = jnp.full((16,), 1.0, jnp.float32)
        return 0
    lax.fori_loop(0, CH // 16, fill, 0)

    pltpu.sync_copy(z1_hbm.at[pl.ds(sid * ROWS_T, ROWS_T)],
                    acc.at[pl.ds(sid * ROWS_T, ROWS_T)])
    plsc.subcore_barrier()

    per_tile = E_PAD // 32
    base = (cid * 16 + sid) * per_tile

    def step(c, _):
        pltpu.sync_copy(dst_hbm.at[pl.ds(base + c * CH, CH)], dst_v)
        pltpu.sync_copy(ones_v, acc.at[dst_v], add=True)
        return 0
    lax.fori_loop(0, per_tile // CH, step, 0)
    plsc.subcore_barrier()

    pltpu.sync_copy(acc.at[pl.ds(sid * ROWS_T, ROWS_T)],
                    out_hbm.at[pl.ds(cid * N_PAD + sid * ROWS_T, ROWS_T)])


def _make_sc_agg(num_groups):
    out_rows = (2 if num_groups == 1 else 4) * N_PAD
    per_tile = E_PAD // 32 if num_groups == 1 else E_PAD // 16

    NBUF = 8
    CHA = 32
    steps = per_tile // CHA
    iters = steps // NBUF

    @functools.partial(
        pl.kernel,
        out_type=jax.ShapeDtypeStruct((out_rows, 128), jnp.float32),
        mesh=_MESH,
        scratch_types=(
            [pltpu.VMEM((per_tile,), jnp.int32)]
            + [pltpu.VMEM((CHA,), jnp.int32) for _ in range(NBUF)]
            + [pltpu.VMEM((CHA, 128), jnp.float32) for _ in range(NBUF)]
            + [pltpu.SemaphoreType.DMA for _ in range(3 * NBUF)]
            + [pltpu.VMEM_SHARED((N_PAD, 128), jnp.float32)]
        ),
    )
    def agg(src_hbm, dst_hbm, tab_hbm, z128_hbm, out_hbm, *refs):
        src_all = refs[0]
        dstb = refs[1:1 + NBUF]
        rows = refs[1 + NBUF:1 + 2 * NBUF]
        dsem = refs[1 + 2 * NBUF:1 + 3 * NBUF]
        gsem = refs[1 + 3 * NBUF:1 + 4 * NBUF]
        ssem = refs[1 + 4 * NBUF:1 + 5 * NBUF]
        acc = refs[1 + 5 * NBUF]
        cid = lax.axis_index("c")
        sid = lax.axis_index("s")
        groups = 1 if num_groups == 1 else 2

        for j in range(groups):
            pltpu.sync_copy(z128_hbm.at[pl.ds(sid * ROWS_T, ROWS_T)],
                            acc.at[pl.ds(sid * ROWS_T, ROWS_T)])

            if num_groups == 1:
                ebase = (cid * 16 + sid) * per_tile
                goff = 0
                orow = cid * N_PAD
            else:
                g = cid * 2 + j
                ebase = sid * per_tile
                goff = g * E_PAD
                orow = g * N_PAD

            pltpu.sync_copy(src_hbm.at[pl.ds(goff + ebase, per_tile)],
                            src_all)
            plsc.subcore_barrier()

            def fire(c, k):
                pltpu.async_copy(dst_hbm.at[pl.ds(ebase + c * CHA, CHA)],
                                 dstb[k], dsem[k])
                pltpu.async_copy(tab_hbm.at[src_all.at[pl.ds(c * CHA, CHA)]],
                                 rows[k], gsem[k])

            def wait_fire(c, k):
                pltpu.make_async_copy(dst_hbm.at[pl.ds(ebase + c * CHA, CHA)],
                                      dstb[k], dsem[k]).wait()
                pltpu.make_async_copy(
                    tab_hbm.at[src_all.at[pl.ds(c * CHA, CHA)]],
                    rows[k], gsem[k]).wait()

            def scatter(k):
                pltpu.async_copy(rows[k], acc.at[dstb[k]], ssem[k],
                                 add=True)

            def wait_scatter(k):
                pltpu.make_async_copy(rows[k], acc.at[dstb[k]],
                                      ssem[k]).wait()

            for k in range(NBUF):
                fire(jnp.int32(k), k)
            for k in range(NBUF):
                wait_fire(jnp.int32(k), k)
                scatter(k)

            def body(c2, _):
                cb = c2 * NBUF
                for k in range(NBUF):
                    wait_scatter(k)
                    fire(cb + k, k)
                for k in range(NBUF):
                    wait_fire(cb + k, k)
                    scatter(k)
                return 0
            lax.fori_loop(1, iters, body, 0)
            for k in range(NBUF):
                wait_scatter(k)
            plsc.subcore_barrier()

            pltpu.sync_copy(acc.at[pl.ds(sid * ROWS_T, ROWS_T)],
                            out_hbm.at[pl.ds(orow + sid * ROWS_T, ROWS_T)])

    return agg


_sc_agg_x = _make_sc_agg(1)
_sc_agg_h = _make_sc_agg(4)


def _tc_prep_body(degt_ref, x_ref, src_ref, dinv_ref, x1_ref, srcg_ref):
    deg = degt_ref[:, 0:1] + degt_ref[:, 1:2] + 1.0
    dinv = lax.rsqrt(deg)
    dinv_ref[...] = dinv
    x1_ref[...] = x_ref[...] * dinv
    s = src_ref[...]
    for g in range(4):
        srcg_ref[g] = s + g * N_PAD


def _tc_prep(degt, x128, src2d):
    return pl.pallas_call(
        _tc_prep_body,
        out_shape=(
            jax.ShapeDtypeStruct((N_PAD, 1), jnp.float32),
            jax.ShapeDtypeStruct((N_PAD, 128), jnp.float32),
            jax.ShapeDtypeStruct((4, E_PAD // 128, 128), jnp.int32),
        ),
    )(degt, x128, src2d)


def _tc_h1_body(aggx_ref, x1_ref, dinv_ref, w1_ref, b1_ref, out_ref):
    dinv = dinv_ref[...]
    agg = aggx_ref[0] + aggx_ref[1] + x1_ref[...]
    t = agg * dinv
    h = jnp.dot(t, w1_ref[...], preferred_element_type=jnp.float32)
    h = jnp.maximum(h + b1_ref[...], 0.0) * dinv
    for g in range(4):
        out_ref[g] = h[:, g * 128:(g + 1) * 128]


def _tc_h1(aggx, x1, dinv, w1p, b1):
    return pl.pallas_call(
        _tc_h1_body,
        grid=(NB,),
        in_specs=[
            pl.BlockSpec((2, 512, 128), lambda i: (0, i, 0)),
            pl.BlockSpec((512, 128), lambda i: (i, 0)),
            pl.BlockSpec((512, 1), lambda i: (i, 0)),
            pl.BlockSpec((128, HIDDEN), lambda i: (0, 0)),
            pl.BlockSpec((1, HIDDEN), lambda i: (0, 0)),
        ],
        out_specs=pl.BlockSpec((4, 512, 128), lambda i: (0, i, 0)),
        out_shape=jax.ShapeDtypeStruct((4, N_PAD, 128), jnp.float32),
        compiler_params=pltpu.CompilerParams(
            dimension_semantics=("arbitrary",)),
    )(aggx, x1, dinv, w1p, b1)


def _tc_out_body(agg_ref, h1g_ref, dinv_ref, batch_ref, w2_ref, b2_ref,
                 wl_ref, bl_ref, out_ref, pool_acc, cnt_acc):
    i = pl.program_id(0)

    @pl.when(i == 0)
    def _():
        pool_acc[...] = jnp.zeros_like(pool_acc)
        cnt_acc[...] = jnp.zeros_like(cnt_acc)

    z = jnp.concatenate(
        [agg_ref[g] + h1g_ref[g] for g in range(4)], axis=1)
    z = z * dinv_ref[...]
    h2 = jnp.dot(z, w2_ref[...], preferred_element_type=jnp.float32)
    h2 = jnp.maximum(h2 + b2_ref[...], 0.0)

    gid = lax.broadcasted_iota(jnp.int32, (512, G), 1)
    p = (batch_ref[...] == gid).astype(jnp.float32)
    pool_acc[...] += lax.dot_general(
        p, h2, (((0,), (0,)), ((), ())),
        preferred_element_type=jnp.float32)
    cnt_acc[...] += lax.dot_general(
        p, jnp.ones((512, 1), jnp.float32), (((0,), (0,)), ((), ())),
        preferred_element_type=jnp.float32)

    @pl.when(i == NB - 1)
    def _():
        pooled = pool_acc[...] / jnp.maximum(cnt_acc[...], 1.0)
        out_ref[...] = (jnp.dot(pooled, wl_ref[...],
                                preferred_element_type=jnp.float32)
                        + bl_ref[...])


def _tc_out(agg, h1g, dinv, batch2d, w2, b2, wl, bl):
    return pl.pallas_call(
        _tc_out_body,
        grid=(NB,),
        in_specs=[
            pl.BlockSpec((4, 512, 128), lambda i: (0, i, 0)),
            pl.BlockSpec((4, 512, 128), lambda i: (0, i, 0)),
            pl.BlockSpec((512, 1), lambda i: (i, 0)),
            pl.BlockSpec((512, 1), lambda i: (i, 0)),
            pl.BlockSpec((HIDDEN, HIDDEN), lambda i: (0, 0)),
            pl.BlockSpec((1, HIDDEN), lambda i: (0, 0)),
            pl.BlockSpec((HIDDEN, OUT_DIM), lambda i: (0, 0)),
            pl.BlockSpec((1, OUT_DIM), lambda i: (0, 0)),
        ],
        out_specs=pl.BlockSpec((G, OUT_DIM), lambda i: (0, 0)),
        out_shape=jax.ShapeDtypeStruct((G, OUT_DIM), jnp.float32),
        scratch_shapes=[
            pltpu.VMEM((G, HIDDEN), jnp.float32),
            pltpu.VMEM((G, 1), jnp.float32),
        ],
        compiler_params=pltpu.CompilerParams(
            dimension_semantics=("arbitrary",)),
    )(agg, h1g, dinv, batch2d, w2, b2, wl, bl)


def kernel(x, edge_index, batch, W1, b1, W2, b2, Wl, bl):
    f32 = jnp.float32
    src = jnp.pad(edge_index[0], (0, E_PAD - E))
    dst = jnp.pad(edge_index[1], (0, E_PAD - E), constant_values=N)
    x128 = jnp.pad(x, ((0, N_PAD - N), (0, 128 - IN_DIM)))
    batch2d = jnp.pad(batch, (0, N_PAD - N), constant_values=G)[:, None]
    w1p = jnp.pad(W1, ((0, 128 - IN_DIM), (0, 0)))
    b1r = b1[None, :]
    b2r = b2[None, :]
    blr = bl[None, :]
    z1 = jnp.zeros((N_PAD,), f32)
    z128 = jnp.zeros((N_PAD, 128), f32)

    degp = _sc_deg(dst, z1)
    degt = jnp.transpose(degp.reshape(2, N_PAD))
    dinv, x1, srcg = _tc_prep(degt, x128, src.reshape(E_PAD // 128, 128))

    aggx = _sc_agg_x(src, dst, x1, z128)
    h1g = _tc_h1(aggx.reshape(2, N_PAD, 128), x1, dinv, w1p, b1r)

    agg1 = _sc_agg_h(srcg.reshape(4 * E_PAD), dst,
                     h1g.reshape(4 * N_PAD, 128), z128)
    return _tc_out(agg1.reshape(4, N_PAD, 128), h1g, dinv, batch2d,
                   W2, b2r, Wl, blr)
